# Initial kernel scaffold; baseline (speedup 1.0000x reference)
import jax
import jax.numpy as jnp
from jax.experimental import pallas as pl


def kernel(feat_prop, neigh_idx, W1, b1, W2, b2, W3, b3):
    raise NotImplementedError("write your pallas kernel here")


# speedup vs baseline: 2.9059x; 2.9059x over previous
import functools

import jax
import jax.numpy as jnp
from jax import lax
from jax.experimental import pallas as pl
from jax.experimental.pallas import tpu as pltpu
from jax.experimental.pallas import tpu_sc as plsc

N = 10000
C = 128
K = 32
E = N * K
NC = 2
NS = 16
NW = NC * NS
EW = E // NW
R = 80
NCHUNK = EW // R


def _sc_gather(table, idx3):
    @functools.partial(
        pl.kernel,
        mesh=plsc.VectorSubcoreMesh(core_axis_name="c", subcore_axis_name="s"),
        out_type=jax.ShapeDtypeStruct((E, C), jnp.float32),
        scratch_types=[
            pltpu.VMEM((NCHUNK, R), jnp.int32),
            pltpu.VMEM((R, C), jnp.float32),
            pltpu.SemaphoreType.DMA,
        ],
    )
    def gather_kernel(table_hbm, idx_hbm, out_hbm, idx_v, rows_v, sem):
        wid = lax.axis_index("s") * NC + lax.axis_index("c")
        pltpu.sync_copy(idx_hbm.at[wid], idx_v)
        base = wid * EW

        def body(c, carry):
            pltpu.async_copy(table_hbm.at[idx_v.at[c]], rows_v, sem).wait()
            pltpu.sync_copy(rows_v, out_hbm.at[pl.ds(base + c * R, R)])
            return carry

        lax.fori_loop(0, NCHUNK, body, 0)

    return gather_kernel(table, idx3)


BN = 200
GRID = N // BN


def _tc_layer_body(g_ref, f_ref, wn_ref, wc_ref, b_ref, o_ref):
    g = g_ref[...]
    f = f_ref[...]
    gw = jnp.dot(g, wn_ref[...], preferred_element_type=jnp.float32)
    cw = jnp.dot(f, wc_ref[...], preferred_element_type=jnp.float32) + b_ref[...]
    g3 = g.reshape(BN, K, C)
    dots = jnp.sum(g3 * f[:, None, :], axis=2)
    ng = jnp.sum(g3 * g3, axis=2)
    nf = jnp.sum(f * f, axis=1, keepdims=True)
    w = dots * lax.rsqrt(ng * nf)
    out = (gw.reshape(BN, K, C) + cw[:, None, :]) * w[:, :, None]
    o_ref[...] = jnp.max(out, axis=1)


def _tc_layer(g, feat, wn, wc, b2d):
    return pl.pallas_call(
        _tc_layer_body,
        grid=(GRID,),
        in_specs=[
            pl.BlockSpec((BN * K, C), lambda i: (i, 0)),
            pl.BlockSpec((BN, C), lambda i: (i, 0)),
            pl.BlockSpec((C, C), lambda i: (0, 0)),
            pl.BlockSpec((C, C), lambda i: (0, 0)),
            pl.BlockSpec((1, C), lambda i: (0, 0)),
        ],
        out_specs=pl.BlockSpec((BN, C), lambda i: (i, 0)),
        out_shape=jax.ShapeDtypeStruct((N, C), jnp.float32),
    )(g, feat, wn, wc, b2d)


def kernel(feat_prop, neigh_idx, W1, b1, W2, b2, W3, b3):
    idx3 = neigh_idx.reshape(NW, NCHUNK, R)
    h = feat_prop
    for W, b in ((W1, b1), (W2, b2), (W3, b3)):
        g = _sc_gather(h, idx3)
        h = _tc_layer(g, h, W[:C], W[C:], b.reshape(1, C))
    return h
